# Initial kernel scaffold; baseline (speedup 1.0000x reference)
import jax
import jax.numpy as jnp
from jax.experimental import pallas as pl


def kernel(x):
    raise NotImplementedError("write your pallas kernel here")


# speedup vs baseline: 1.1171x; 1.1171x over previous
import functools

import jax
import jax.numpy as jnp
from jax import lax
from jax.experimental import pallas as pl
from jax.experimental.pallas import tpu as pltpu

_TINY = float(jnp.finfo(jnp.float32).tiny)
_KS0 = 0
_KS1 = 42
_KS2 = 0x1BD11BDA ^ 0 ^ 42
_ROT = ((13, 15, 26, 6), (17, 29, 16, 24))


def _i32(v):
    return jnp.int32(jnp.uint32(v))


def _threefry_bits(x1):
    ks = (_i32(_KS0), _i32(_KS1), _i32(_KS2))
    x0 = jnp.zeros_like(x1) + ks[0]
    x1 = x1 + ks[1]
    for i in range(5):
        for r in _ROT[i % 2]:
            x0 = x0 + x1
            x1 = (lax.shift_left(x1, jnp.int32(r))
                  | lax.shift_right_logical(x1, jnp.int32(32 - r)))
            x1 = lax.bitwise_xor(x0, x1)
        x0 = x0 + ks[(i + 1) % 3]
        x1 = x1 + ks[(i + 2) % 3] + _i32(i + 1)
    return lax.bitwise_xor(x0, x1)


def _score(v, f):
    bits = _threefry_bits(f)
    fb = lax.bitwise_or(lax.shift_right_logical(bits, jnp.int32(9)),
                        jnp.int32(0x3F800000))
    u = lax.bitcast_convert_type(fb, jnp.float32) - jnp.float32(1.0)
    u = jnp.maximum(u, jnp.float32(_TINY))
    return v / (-jnp.log(u))


def _mc_pool_kernel(xe_ref, xo_ref, o_ref, *, cb, h, w):
    ch0 = pl.program_id(0) * cb
    ph, pw = h // 2, w // 2
    ve = xe_ref[:, :, 0, 0, :]
    vo = xo_ref[:, :, 0, 0, :]

    ch = (lax.broadcasted_iota(jnp.int32, (cb, ph, w), 0) + ch0) * _i32(h * w)
    i = lax.broadcasted_iota(jnp.int32, (cb, ph, w), 1)
    c = lax.broadcasted_iota(jnp.int32, (cb, ph, w), 2)
    f = (ch + i * _i32(4 * pw) + c * _i32(2) - lax.bitwise_and(c, jnp.int32(1)))

    s0 = _score(ve, f)
    s1 = _score(vo, f + _i32(2))
    s0r = pltpu.roll(s0, w - 1, 2)
    s1r = pltpu.roll(s1, w - 1, 2)

    best = s0
    idx = jnp.zeros_like(s0)
    idx = jnp.where(s0r > best, jnp.float32(1.0), idx)
    best = jnp.maximum(best, s0r)
    idx = jnp.where(s1 > best, jnp.float32(2.0), idx)
    best = jnp.maximum(best, s1)
    idx = jnp.where(s1r > best, jnp.float32(3.0), idx)

    sel = (lax.broadcasted_iota(jnp.int32, (w, pw), 0)
           == lax.broadcasted_iota(jnp.int32, (w, pw), 1) * 2
           ).astype(jnp.float32)
    for b in range(cb):
        o_ref[b] = jnp.dot(idx[b], sel)


def kernel(x):
    batch, chan, h, w = x.shape
    nch = batch * chan
    cb = 4
    xr = x.reshape(nch, h // 2, 2, 1, w)
    grid = (nch // cb,)
    out = pl.pallas_call(
        functools.partial(_mc_pool_kernel, cb=cb, h=h, w=w),
        grid=grid,
        in_specs=[
            pl.BlockSpec((cb, h // 2, 1, 1, w), lambda i: (i, 0, 0, 0, 0)),
            pl.BlockSpec((cb, h // 2, 1, 1, w), lambda i: (i, 0, 1, 0, 0)),
        ],
        out_specs=pl.BlockSpec((cb, h // 2, w // 2), lambda i: (i, 0, 0)),
        out_shape=jax.ShapeDtypeStruct((nch, h // 2, w // 2), jnp.float32),
    )(xr, xr)
    return out.reshape(batch, chan, h // 2, w // 2)
